# baseline (speedup 1.0000x reference)
import functools

import jax
import jax.numpy as jnp
from jax import lax
from jax.experimental import pallas as pl
from jax.experimental.pallas import tpu as pltpu
from jax.experimental.pallas import tpu_sc as plsc

_B, _T, _NEOF, _EMB = 4, 2048, 128, 32
_NW = 32
_ROWS = (_B * _T) // _NW
_L = 16
_NJ = _NEOF // _L
_NK = _ROWS // _L
_EXT = _EMB + 1


def _bcast(ref, idx):
    return plsc.load_gather(ref, [jnp.full((_L,), idx, jnp.int32)])


def _sc_body(x_hbm, tt_hbm, ft_hbm, w_hbm, out_hbm, x_v, tt_v, ft_v, w_v, tp_v):
    wid = lax.axis_index("s") * 2 + lax.axis_index("c")
    base = wid * _ROWS

    pltpu.sync_copy(x_hbm.at[pl.ds(base, _ROWS)], x_v)
    pltpu.sync_copy(tt_hbm.at[wid % (_T // _ROWS)], tt_v)
    pltpu.sync_copy(ft_hbm, ft_v)
    pltpu.sync_copy(w_hbm, w_v)

    facc = [jnp.zeros((_L,), jnp.float32) for _ in range(_NJ)]
    tacc = [jnp.zeros((_L,), jnp.float32) for _ in range(_NK)]
    for e in range(_EXT):
        we = _bcast(w_v, e + 1)
        for j in range(_NJ):
            facc[j] = facc[j] + we * ft_v[e, pl.ds(j * _L, _L)]
        if e < _EMB:
            for k in range(_NK):
                tacc[k] = tacc[k] + we * tt_v[e, pl.ds(k * _L, _L)]
    for k in range(_NK):
        tp_v[pl.ds(k * _L, _L)] = tacc[k]

    def row_step(r, carry):
        tpb = _bcast(tp_v, r)
        for j in range(_NJ):
            sl = pl.ds(j * _L, _L)
            x_v[r, sl] = x_v[r, sl] + (facc[j] + tpb)
        return carry

    lax.fori_loop(0, _ROWS, row_step, 0)
    pltpu.sync_copy(x_v, out_hbm.at[pl.ds(base, _ROWS)])


def kernel(X, time_table, feature_table, W, b):
    Xf = X.reshape(_B * _T, _NEOF)
    tt_t = time_table.T.reshape(_EMB, _T // _ROWS, _ROWS).transpose(1, 0, 2)
    ft_ext = jnp.concatenate(
        [feature_table.T, jnp.ones((1, _NEOF), jnp.float32)], axis=0)
    w_ext = jnp.concatenate(
        [jnp.zeros((1,), jnp.float32), W.reshape(_EMB), b,
         jnp.zeros((3 * _L - _EXT - 1,), jnp.float32)])

    mesh = plsc.VectorSubcoreMesh(core_axis_name="c", subcore_axis_name="s")
    run = pl.kernel(
        _sc_body,
        mesh=mesh,
        out_type=jax.ShapeDtypeStruct((_B * _T, _NEOF), jnp.float32),
        scratch_types=[
            pltpu.VMEM((_ROWS, _NEOF), jnp.float32),
            pltpu.VMEM((_EMB, _ROWS), jnp.float32),
            pltpu.VMEM((_EXT, _NEOF), jnp.float32),
            pltpu.VMEM((3 * _L,), jnp.float32),
            pltpu.VMEM((_ROWS,), jnp.float32),
        ],
        compiler_params=pltpu.CompilerParams(needs_layout_passes=False),
    )
    out = run(Xf, tt_t, ft_ext, w_ext)
    return out.reshape(_B, _T, _NEOF)
